# baseline (speedup 1.0000x reference)
import functools

import jax
import jax.numpy as jnp
from jax import lax
from jax.experimental import pallas as pl
from jax.experimental.pallas import tpu as pltpu
from jax.experimental.pallas import tpu_sc as plsc

N = 10000
E = 320000
D = 128
TD = 32
DM = 64
NC = 32
NF = 16

NUM_SC_CORES = 2
NUM_SUBCORES = 16
NW = NUM_SC_CORES * NUM_SUBCORES
CH = E // NW
SEG = CH // 16
LCAP = 8
SLOTS = 16 * LCAP
K = NW * SLOTS


def _sc_body(ei_hbm, t_hbm, mask_hbm, lu_hbm, msg_hbm, z_hbm,
             ts_hbm, td_hbm, tgt_hbm,
             wts_out, wtd_out, rel_out, zr_out, mr_out, ztgt_out,
             dst_v, src_v, t_v, mask_v, lu_v, ts_v, td_v, tgt_v,
             ids_v, gid_v, sid_v, zrow_v, mrow_v, ztgt_v,
             wts_v, wtd_v, rel_v, sem):
    wid = lax.axis_index("s") * NUM_SC_CORES + lax.axis_index("c")
    base = wid * CH

    pltpu.sync_copy(ei_hbm.at[1, pl.ds(base, CH)], dst_v)
    pltpu.sync_copy(ei_hbm.at[0, pl.ds(base, CH)], src_v)
    pltpu.sync_copy(t_hbm.at[pl.ds(base, CH)], t_v)
    pltpu.sync_copy(mask_hbm.at[pl.ds(base, CH)], mask_v)
    pltpu.sync_copy(lu_hbm, lu_v)
    pltpu.sync_copy(ts_hbm, ts_v)
    pltpu.sync_copy(td_hbm, td_v)

    iota16 = lax.iota(jnp.int32, 16)
    tsv = ts_v[...]
    tdv = td_v[...]
    seg_base = iota16 * SEG
    lane_base = iota16 * LCAP

    def body(i, cnt):
        eid = seg_base + i
        d = plsc.load_gather(dst_v, [eid])
        m = (d == tsv) | (d == tdv)
        pos = lane_base + jnp.minimum(cnt, LCAP - 1)
        plsc.store_scatter(ids_v, [pos], eid, mask=m)
        return cnt + jnp.where(m, 1, 0).astype(jnp.int32)

    cnt = lax.fori_loop(0, SEG, body, jnp.zeros((16,), jnp.int32))

    for b in range(LCAP):
        valid = cnt > b
        ids_raw = plsc.load_gather(ids_v, [lane_base + b])
        ids = jnp.maximum(jnp.minimum(ids_raw, CH - 1), 0)
        d = plsc.load_gather(dst_v, [ids])
        s = plsc.load_gather(src_v, [ids])
        tv = plsc.load_gather(t_v, [ids])
        mv = plsc.load_gather(mask_v, [ids])
        lu = plsc.load_gather(lu_v, [s])
        relv = tv - lu
        msv = 1.0 / (1.0 + jnp.exp(-mv))
        zero = jnp.zeros((16,), jnp.float32)
        wts = jnp.where(valid & (d == tsv), msv, zero)
        wtd = jnp.where(valid & (d == tdv), msv, zero)
        wts_v[pl.ds(b * 16, 16)] = wts
        wtd_v[pl.ds(b * 16, 16)] = wtd
        rel_v[pl.ds(b * 16, 16)] = relv
        gid_v[...] = base + ids
        sid_v[...] = s
        pltpu.async_copy(z_hbm.at[sid_v], zrow_v, sem).wait()
        pltpu.async_copy(msg_hbm.at[gid_v], mrow_v, sem).wait()
        pltpu.sync_copy(zrow_v, zr_out.at[pl.ds(wid * SLOTS + b * 16, 16)])
        pltpu.sync_copy(mrow_v, mr_out.at[pl.ds(wid * SLOTS + b * 16, 16)])

    pltpu.sync_copy(wts_v, wts_out.at[pl.ds(wid * SLOTS, SLOTS)])
    pltpu.sync_copy(wtd_v, wtd_out.at[pl.ds(wid * SLOTS, SLOTS)])
    pltpu.sync_copy(rel_v, rel_out.at[pl.ds(wid * SLOTS, SLOTS)])

    @pl.when(wid == 0)
    def _():
        pltpu.sync_copy(tgt_hbm, tgt_v)
        pltpu.async_copy(z_hbm.at[tgt_v], ztgt_v, sem).wait()
        pltpu.sync_copy(ztgt_v, ztgt_out)


_sc_find = functools.partial(
    pl.kernel,
    mesh=plsc.VectorSubcoreMesh(core_axis_name="c", subcore_axis_name="s"),
    compiler_params=pltpu.CompilerParams(
        needs_layout_passes=False, use_tc_tiling_on_sc=False),
    out_type=[
        jax.ShapeDtypeStruct((K,), jnp.float32),
        jax.ShapeDtypeStruct((K,), jnp.float32),
        jax.ShapeDtypeStruct((K,), jnp.float32),
        jax.ShapeDtypeStruct((K, D), jnp.float32),
        jax.ShapeDtypeStruct((K, DM), jnp.float32),
        jax.ShapeDtypeStruct((16, D), jnp.float32),
    ],
    scratch_types=[
        pltpu.VMEM((CH,), jnp.int32),
        pltpu.VMEM((CH,), jnp.int32),
        pltpu.VMEM((CH,), jnp.float32),
        pltpu.VMEM((CH,), jnp.float32),
        pltpu.VMEM((N,), jnp.float32),
        pltpu.VMEM((16,), jnp.int32),
        pltpu.VMEM((16,), jnp.int32),
        pltpu.VMEM((16,), jnp.int32),
        pltpu.VMEM((SLOTS,), jnp.int32),
        pltpu.VMEM((16,), jnp.int32),
        pltpu.VMEM((16,), jnp.int32),
        pltpu.VMEM((16, D), jnp.float32),
        pltpu.VMEM((16, DM), jnp.float32),
        pltpu.VMEM((16, D), jnp.float32),
        pltpu.VMEM((SLOTS,), jnp.float32),
        pltpu.VMEM((SLOTS,), jnp.float32),
        pltpu.VMEM((SLOTS,), jnp.float32),
        pltpu.SemaphoreType.DMA,
    ],
)(_sc_body)


def _tc_body(mask_ref, wts_ref, wtd_ref, rel_ref, zr_ref, mr_ref, ztgt_ref,
             wtime_ref, wmsg_ref, bmsg_ref, wupd_ref, bupd_ref,
             w1_ref, b1_ref, w2_ref, b2_ref, msg0_ref, out_ref):
    x = mask_ref[...].reshape(E // 128, 128)
    ms = 1.0 / (1.0 + jnp.exp(-x))
    sum_ms = jnp.sum(ms)
    ent = jnp.sum(ms * jnp.log(ms + 1e-8) + (1.0 - ms) * jnp.log(1.0 - ms + 1e-8))

    te = jnp.cos(rel_ref[...] * wtime_ref[...])
    xcat = jnp.concatenate([zr_ref[...], te, mr_ref[...]], axis=1)
    m = jnp.dot(xcat, wmsg_ref[...], preferred_element_type=jnp.float32)
    m = jnp.maximum(m + bmsg_ref[...], 0.0)

    agg_s = jnp.dot(wts_ref[...], m, preferred_element_type=jnp.float32)
    agg_d = jnp.dot(wtd_ref[...], m, preferred_element_type=jnp.float32)

    z_s_in = jnp.concatenate([ztgt_ref[0:1, :], agg_s], axis=1)
    z_d_in = jnp.concatenate([ztgt_ref[1:2, :], agg_d], axis=1)
    z_s = jnp.maximum(jnp.dot(z_s_in, wupd_ref[...],
                              preferred_element_type=jnp.float32) + bupd_ref[...], 0.0)
    z_d = jnp.maximum(jnp.dot(z_d_in, wupd_ref[...],
                              preferred_element_type=jnp.float32) + bupd_ref[...], 0.0)

    h = jnp.concatenate([z_s, z_d], axis=1)
    h = jnp.maximum(jnp.dot(h, w1_ref[...],
                            preferred_element_type=jnp.float32) + b1_ref[...], 0.0)
    logits = jnp.dot(h, w2_ref[...],
                     preferred_element_type=jnp.float32) + b2_ref[...]

    feat = msg0_ref[:, NF:NF + NC]
    iota2 = lax.broadcasted_iota(jnp.int32, (1, NC), 1)
    mx = jnp.max(feat)
    lbl = jnp.min(jnp.where(feat == mx, iota2, NC))
    logit_lbl = jnp.sum(jnp.where(iota2 == lbl, logits, 0.0))

    lm = jnp.max(logits)
    lse = lm + jnp.log(jnp.sum(jnp.exp(logits - lm)))
    loss_pred = lse - logit_lbl

    total = loss_pred + 0.005 * sum_ms - 0.01 * ent
    out_ref[0, 0] = total


def kernel(z_original, last_update, edge_index, subgraph_t, subgraph_msg,
           edge_mask, w_time, W_msg, b_msg, W_upd, b_upd, W1, b1, W2, b2,
           target_src_local, target_dst_local):
    ts = jnp.asarray(target_src_local, jnp.int32)
    td = jnp.asarray(target_dst_local, jnp.int32)
    ts16 = jnp.full((16,), ts, jnp.int32)
    td16 = jnp.full((16,), td, jnp.int32)
    tgt16 = jnp.concatenate([ts[None], td[None], jnp.zeros((14,), jnp.int32)])

    wts, wtd, rel, zrows, mrows, ztgt = _sc_find(
        edge_index, subgraph_t, edge_mask, last_update, subgraph_msg,
        z_original, ts16, td16, tgt16)

    total = pl.pallas_call(
        _tc_body,
        out_shape=jax.ShapeDtypeStruct((1, 1), jnp.float32),
        out_specs=pl.BlockSpec(memory_space=pltpu.SMEM),
    )(
        edge_mask,
        wts.reshape(1, K),
        wtd.reshape(1, K),
        rel.reshape(K, 1),
        zrows,
        mrows,
        ztgt,
        w_time.reshape(1, TD),
        W_msg,
        b_msg.reshape(1, D),
        W_upd,
        b_upd.reshape(1, D),
        W1,
        b1.reshape(1, D),
        W2,
        b2.reshape(1, NC),
        subgraph_msg[0:1, :],
    )
    return total[0, 0]


# speedup vs baseline: 1.0433x; 1.0433x over previous
import functools

import jax
import jax.numpy as jnp
from jax import lax
from jax.experimental import pallas as pl
from jax.experimental.pallas import tpu as pltpu
from jax.experimental.pallas import tpu_sc as plsc

N = 10000
E = 320000
D = 128
TD = 32
DM = 64
NC = 32
NF = 16

NUM_SC_CORES = 2
NUM_SUBCORES = 16
NW = NUM_SC_CORES * NUM_SUBCORES
CH = E // NW
SEG = CH // 16
LCAP = 8
SLOTS = 16 * LCAP
K = NW * SLOTS


def _sc_body(ei_hbm, t_hbm, mask_hbm, lu_hbm, msg_hbm, z_hbm,
             ts_hbm, td_hbm, tgt_hbm,
             wts_out, wtd_out, rel_out, zr_out, mr_out, ztgt_out,
             dst_v, src_v, t_v, mask_v, lu_v, ts_v, td_v, tgt_v,
             ids_v, gid_v, sid_v, zrow_v, mrow_v, ztgt_v,
             wts_v, wtd_v, rel_v, sem, sem2):
    wid = lax.axis_index("s") * NUM_SC_CORES + lax.axis_index("c")
    base = wid * CH

    pltpu.sync_copy(ei_hbm.at[1, pl.ds(base, CH)], dst_v)
    pltpu.sync_copy(ei_hbm.at[0, pl.ds(base, CH)], src_v)
    pltpu.sync_copy(t_hbm.at[pl.ds(base, CH)], t_v)
    pltpu.sync_copy(mask_hbm.at[pl.ds(base, CH)], mask_v)
    pltpu.sync_copy(lu_hbm, lu_v)
    pltpu.sync_copy(ts_hbm, ts_v)
    pltpu.sync_copy(td_hbm, td_v)

    iota16 = lax.iota(jnp.int32, 16)
    tsv = ts_v[...]
    tdv = td_v[...]
    seg_base = iota16 * SEG
    lane_base = iota16 * LCAP

    def body(i, cnt):
        eid = seg_base + i
        d = plsc.load_gather(dst_v, [eid])
        m = (d == tsv) | (d == tdv)
        pos = lane_base + jnp.minimum(cnt, LCAP - 1)
        plsc.store_scatter(ids_v, [pos], eid, mask=m)
        return cnt + jnp.where(m, 1, 0).astype(jnp.int32)

    cnt = lax.fori_loop(0, SEG, body, jnp.zeros((16,), jnp.int32))

    for b in range(LCAP):
        valid = cnt > b
        ids_raw = plsc.load_gather(ids_v, [lane_base + b])
        ids = jnp.maximum(jnp.minimum(ids_raw, CH - 1), 0)
        d = plsc.load_gather(dst_v, [ids])
        s = plsc.load_gather(src_v, [ids])
        tv = plsc.load_gather(t_v, [ids])
        mv = plsc.load_gather(mask_v, [ids])
        lu = plsc.load_gather(lu_v, [s])
        relv = tv - lu
        msv = 1.0 / (1.0 + jnp.exp(-mv))
        zero = jnp.zeros((16,), jnp.float32)
        wts = jnp.where(valid & (d == tsv), msv, zero)
        wtd = jnp.where(valid & (d == tdv), msv, zero)
        wts_v[pl.ds(b * 16, 16)] = wts
        wtd_v[pl.ds(b * 16, 16)] = wtd
        rel_v[pl.ds(b * 16, 16)] = relv
        gid_v[pl.ds(b * 16, 16)] = base + ids
        sid_v[pl.ds(b * 16, 16)] = s

    cz = pltpu.async_copy(z_hbm.at[sid_v], zrow_v, sem)
    cm = pltpu.async_copy(msg_hbm.at[gid_v], mrow_v, sem2)
    cz.wait()
    cm.wait()
    pltpu.sync_copy(zrow_v, zr_out.at[pl.ds(wid * SLOTS, SLOTS)])
    pltpu.sync_copy(mrow_v, mr_out.at[pl.ds(wid * SLOTS, SLOTS)])
    pltpu.sync_copy(wts_v, wts_out.at[pl.ds(wid * SLOTS, SLOTS)])
    pltpu.sync_copy(wtd_v, wtd_out.at[pl.ds(wid * SLOTS, SLOTS)])
    pltpu.sync_copy(rel_v, rel_out.at[pl.ds(wid * SLOTS, SLOTS)])

    @pl.when(wid == 0)
    def _():
        pltpu.sync_copy(tgt_hbm, tgt_v)
        pltpu.async_copy(z_hbm.at[tgt_v], ztgt_v, sem).wait()
        pltpu.sync_copy(ztgt_v, ztgt_out)


_sc_find = functools.partial(
    pl.kernel,
    mesh=plsc.VectorSubcoreMesh(core_axis_name="c", subcore_axis_name="s"),
    compiler_params=pltpu.CompilerParams(
        needs_layout_passes=False, use_tc_tiling_on_sc=False),
    out_type=[
        jax.ShapeDtypeStruct((K,), jnp.float32),
        jax.ShapeDtypeStruct((K,), jnp.float32),
        jax.ShapeDtypeStruct((K,), jnp.float32),
        jax.ShapeDtypeStruct((K, D), jnp.float32),
        jax.ShapeDtypeStruct((K, DM), jnp.float32),
        jax.ShapeDtypeStruct((16, D), jnp.float32),
    ],
    scratch_types=[
        pltpu.VMEM((CH,), jnp.int32),
        pltpu.VMEM((CH,), jnp.int32),
        pltpu.VMEM((CH,), jnp.float32),
        pltpu.VMEM((CH,), jnp.float32),
        pltpu.VMEM((N,), jnp.float32),
        pltpu.VMEM((16,), jnp.int32),
        pltpu.VMEM((16,), jnp.int32),
        pltpu.VMEM((16,), jnp.int32),
        pltpu.VMEM((SLOTS,), jnp.int32),
        pltpu.VMEM((SLOTS,), jnp.int32),
        pltpu.VMEM((SLOTS,), jnp.int32),
        pltpu.VMEM((SLOTS, D), jnp.float32),
        pltpu.VMEM((SLOTS, DM), jnp.float32),
        pltpu.VMEM((16, D), jnp.float32),
        pltpu.VMEM((SLOTS,), jnp.float32),
        pltpu.VMEM((SLOTS,), jnp.float32),
        pltpu.VMEM((SLOTS,), jnp.float32),
        pltpu.SemaphoreType.DMA,
        pltpu.SemaphoreType.DMA,
    ],
)(_sc_body)


def _tc_mask_body(mask_ref, out_ref):
    x = mask_ref[...].reshape(E // 128, 128)
    ms = 1.0 / (1.0 + jnp.exp(-x))
    out_ref[0, 0] = jnp.sum(ms)
    out_ref[0, 1] = jnp.sum(
        ms * jnp.log(ms + 1e-8) + (1.0 - ms) * jnp.log(1.0 - ms + 1e-8))


def _tc_body(sums_ref, wts_ref, wtd_ref, rel_ref, zr_ref, mr_ref, ztgt_ref,
             wtime_ref, wmsg_ref, bmsg_ref, wupd_ref, bupd_ref,
             w1_ref, b1_ref, w2_ref, b2_ref, msg0_ref, out_ref):
    sv = sums_ref[...]
    sum_ms = jnp.sum(sv[0:1, 0:1])
    ent = jnp.sum(sv[0:1, 1:2])

    te = jnp.cos(rel_ref[...] * wtime_ref[...])
    xcat = jnp.concatenate([zr_ref[...], te, mr_ref[...]], axis=1)
    m = jnp.dot(xcat, wmsg_ref[...], preferred_element_type=jnp.float32)
    m = jnp.maximum(m + bmsg_ref[...], 0.0)

    agg_s = jnp.dot(wts_ref[...], m, preferred_element_type=jnp.float32)
    agg_d = jnp.dot(wtd_ref[...], m, preferred_element_type=jnp.float32)

    z_s_in = jnp.concatenate([ztgt_ref[0:1, :], agg_s], axis=1)
    z_d_in = jnp.concatenate([ztgt_ref[1:2, :], agg_d], axis=1)
    z_s = jnp.maximum(jnp.dot(z_s_in, wupd_ref[...],
                              preferred_element_type=jnp.float32) + bupd_ref[...], 0.0)
    z_d = jnp.maximum(jnp.dot(z_d_in, wupd_ref[...],
                              preferred_element_type=jnp.float32) + bupd_ref[...], 0.0)

    h = jnp.concatenate([z_s, z_d], axis=1)
    h = jnp.maximum(jnp.dot(h, w1_ref[...],
                            preferred_element_type=jnp.float32) + b1_ref[...], 0.0)
    logits = jnp.dot(h, w2_ref[...],
                     preferred_element_type=jnp.float32) + b2_ref[...]

    feat = msg0_ref[:, NF:NF + NC]
    iota2 = lax.broadcasted_iota(jnp.int32, (1, NC), 1)
    mx = jnp.max(feat)
    lbl = jnp.min(jnp.where(feat == mx, iota2, NC))
    logit_lbl = jnp.sum(jnp.where(iota2 == lbl, logits, 0.0))

    lm = jnp.max(logits)
    lse = lm + jnp.log(jnp.sum(jnp.exp(logits - lm)))
    loss_pred = lse - logit_lbl

    total = loss_pred + 0.005 * sum_ms - 0.01 * ent
    out_ref[0, 0] = total


def kernel(z_original, last_update, edge_index, subgraph_t, subgraph_msg,
           edge_mask, w_time, W_msg, b_msg, W_upd, b_upd, W1, b1, W2, b2,
           target_src_local, target_dst_local):
    ts = jnp.asarray(target_src_local, jnp.int32)
    td = jnp.asarray(target_dst_local, jnp.int32)
    ts16 = jnp.full((16,), ts, jnp.int32)
    td16 = jnp.full((16,), td, jnp.int32)
    tgt16 = jnp.concatenate([ts[None], td[None], jnp.zeros((14,), jnp.int32)])

    wts, wtd, rel, zrows, mrows, ztgt = _sc_find(
        edge_index, subgraph_t, edge_mask, last_update, subgraph_msg,
        z_original, ts16, td16, tgt16)

    sums = pl.pallas_call(
        _tc_mask_body,
        out_shape=jax.ShapeDtypeStruct((1, 2), jnp.float32),
        out_specs=pl.BlockSpec(memory_space=pltpu.SMEM),
    )(edge_mask)

    total = pl.pallas_call(
        _tc_body,
        out_shape=jax.ShapeDtypeStruct((1, 1), jnp.float32),
        out_specs=pl.BlockSpec(memory_space=pltpu.SMEM),
    )(
        sums,
        wts.reshape(1, K),
        wtd.reshape(1, K),
        rel.reshape(K, 1),
        zrows,
        mrows,
        ztgt,
        w_time.reshape(1, TD),
        W_msg,
        b_msg.reshape(1, D),
        W_upd,
        b_upd.reshape(1, D),
        W1,
        b1.reshape(1, D),
        W2,
        b2.reshape(1, NC),
        subgraph_msg[0:1, :],
    )
    return total[0, 0]


# speedup vs baseline: 1.6072x; 1.5406x over previous
import functools

import jax
import jax.numpy as jnp
from jax import lax
from jax.experimental import pallas as pl
from jax.experimental.pallas import tpu as pltpu
from jax.experimental.pallas import tpu_sc as plsc

N = 10000
E = 320000
D = 128
TD = 32
DM = 64
NC = 32
NF = 16

NUM_SC_CORES = 2
NUM_SUBCORES = 16
NW = NUM_SC_CORES * NUM_SUBCORES
CH = E // NW
SEG = CH // 16
LCAP = 8
SLOTS = 16 * LCAP
SCAP = 16
TOT = NW * SCAP


def _sc_body(src_hbm, dst_hbm, t_hbm, mask_hbm, lu_hbm, ts_hbm, td_hbm,
             cnt_out, sid_out, gid_out, wts_out, wtd_out, rel_out,
             dst_v, src_v, t_v, mask_v, lu_v, ts_v, td_v,
             ids_v, pfx_v, cnt16_v,
             sids_v, gids_v, wtss_v, wtds_v, rels_v):
    wid = lax.axis_index("s") * NUM_SC_CORES + lax.axis_index("c")
    base = wid * CH

    pltpu.sync_copy(dst_hbm.at[pl.ds(base, CH)], dst_v)
    pltpu.sync_copy(src_hbm.at[pl.ds(base, CH)], src_v)
    pltpu.sync_copy(t_hbm.at[pl.ds(base, CH)], t_v)
    pltpu.sync_copy(mask_hbm.at[pl.ds(base, CH)], mask_v)
    pltpu.sync_copy(lu_hbm, lu_v)
    pltpu.sync_copy(ts_hbm, ts_v)
    pltpu.sync_copy(td_hbm, td_v)

    iota16 = lax.iota(jnp.int32, 16)
    tsv = ts_v[...]
    tdv = td_v[...]
    seg_base = iota16 * SEG
    lane_base = iota16 * LCAP

    def body(i, cnt):
        eid = seg_base + i
        d = plsc.load_gather(dst_v, [eid])
        m = (d == tsv) | (d == tdv)
        pos = lane_base + jnp.minimum(cnt, LCAP - 1)
        plsc.store_scatter(ids_v, [pos], eid, mask=m)
        return cnt + jnp.where(m, 1, 0).astype(jnp.int32)

    cnt = lax.fori_loop(0, SEG, body, jnp.zeros((16,), jnp.int32))
    cl = jnp.minimum(cnt, LCAP)

    pfx = cl
    for k in (1, 2, 4, 8):
        pfx_v[...] = pfx
        sh = plsc.load_gather(pfx_v, [jnp.maximum(iota16 - k, 0)])
        pfx = pfx + jnp.where(iota16 >= k, sh, 0)
    excl = pfx - cl
    pfx_v[...] = pfx
    tot = plsc.load_gather(pfx_v, [jnp.full((16,), 15, jnp.int32)])
    cnt16_v[...] = jnp.minimum(tot, SCAP)
    pltpu.sync_copy(cnt16_v, cnt_out.at[wid])

    zero_i = jnp.zeros((SCAP,), jnp.int32)
    zero_f = jnp.zeros((SCAP,), jnp.float32)
    sids_v[...] = zero_i
    gids_v[...] = zero_i
    wtss_v[...] = zero_f
    wtds_v[...] = zero_f
    rels_v[...] = zero_f

    for b in range(LCAP):
        valid = cnt > b
        ids_raw = plsc.load_gather(ids_v, [lane_base + b])
        ids = jnp.maximum(jnp.minimum(ids_raw, CH - 1), 0)
        d = plsc.load_gather(dst_v, [ids])
        s = plsc.load_gather(src_v, [ids])
        tv = plsc.load_gather(t_v, [ids])
        mv = plsc.load_gather(mask_v, [ids])
        lu = plsc.load_gather(lu_v, [s])
        relv = tv - lu
        msv = 1.0 / (1.0 + jnp.exp(-mv))
        zero = jnp.zeros((16,), jnp.float32)
        wts = jnp.where(valid & (d == tsv), msv, zero)
        wtd = jnp.where(valid & (d == tdv), msv, zero)
        pos = jnp.minimum(excl + b, SCAP - 1)
        plsc.store_scatter(sids_v, [pos], s, mask=valid)
        plsc.store_scatter(gids_v, [pos], base + ids, mask=valid)
        plsc.store_scatter(wtss_v, [pos], wts, mask=valid)
        plsc.store_scatter(wtds_v, [pos], wtd, mask=valid)
        plsc.store_scatter(rels_v, [pos], relv, mask=valid)

    pltpu.sync_copy(sids_v, sid_out.at[pl.ds(wid * SCAP, SCAP)])
    pltpu.sync_copy(gids_v, gid_out.at[pl.ds(wid * SCAP, SCAP)])
    pltpu.sync_copy(wtss_v, wts_out.at[pl.ds(wid * SCAP, SCAP)])
    pltpu.sync_copy(wtds_v, wtd_out.at[pl.ds(wid * SCAP, SCAP)])
    pltpu.sync_copy(rels_v, rel_out.at[pl.ds(wid * SCAP, SCAP)])


_sc_find = functools.partial(
    pl.kernel,
    mesh=plsc.VectorSubcoreMesh(core_axis_name="c", subcore_axis_name="s"),
    compiler_params=pltpu.CompilerParams(
        needs_layout_passes=False, use_tc_tiling_on_sc=False),
    out_type=[
        jax.ShapeDtypeStruct((NW, 16), jnp.int32),
        jax.ShapeDtypeStruct((TOT,), jnp.int32),
        jax.ShapeDtypeStruct((TOT,), jnp.int32),
        jax.ShapeDtypeStruct((TOT,), jnp.float32),
        jax.ShapeDtypeStruct((TOT,), jnp.float32),
        jax.ShapeDtypeStruct((TOT,), jnp.float32),
    ],
    scratch_types=[
        pltpu.VMEM((CH,), jnp.int32),
        pltpu.VMEM((CH,), jnp.int32),
        pltpu.VMEM((CH,), jnp.float32),
        pltpu.VMEM((CH,), jnp.float32),
        pltpu.VMEM((N,), jnp.float32),
        pltpu.VMEM((16,), jnp.int32),
        pltpu.VMEM((16,), jnp.int32),
        pltpu.VMEM((SLOTS,), jnp.int32),
        pltpu.VMEM((16,), jnp.int32),
        pltpu.VMEM((16,), jnp.int32),
        pltpu.VMEM((SCAP,), jnp.int32),
        pltpu.VMEM((SCAP,), jnp.int32),
        pltpu.VMEM((SCAP,), jnp.float32),
        pltpu.VMEM((SCAP,), jnp.float32),
        pltpu.VMEM((SCAP,), jnp.float32),
    ],
)(_sc_body)


def _tc_mask_body(mask_ref, out_ref):
    x = mask_ref[...].reshape(E // 128, 128)
    ms = 1.0 / (1.0 + jnp.exp(-x))
    out_ref[0, 0] = jnp.sum(ms)
    out_ref[0, 1] = jnp.sum(
        ms * jnp.log(ms + 1e-8) + (1.0 - ms) * jnp.log(1.0 - ms + 1e-8))


def _tc_body(sums_ref, cnt_ref, sid_ref, gid_ref, tstd_ref,
             wts_ref, wtd_ref, rel_ref, z_any, msg_any,
             wtime_ref, wmsg_ref, bmsg_ref, wupd_ref, bupd_ref,
             w1_ref, b1_ref, w2_ref, b2_ref, msg0_ref, out_ref,
             ztab, mtab, ztgt, sem_z, sem_m):
    ztab[...] = jnp.zeros((TOT, D), jnp.float32)
    mtab[...] = jnp.zeros((TOT, DM), jnp.float32)

    c0 = pltpu.make_async_copy(
        z_any.at[pl.ds(tstd_ref[0], 1)], ztgt.at[pl.ds(0, 1)], sem_z)
    c1 = pltpu.make_async_copy(
        z_any.at[pl.ds(tstd_ref[1], 1)], ztgt.at[pl.ds(1, 1)], sem_z)
    c0.start()
    c1.start()

    def outer_start(w, carry):
        c = cnt_ref[w, 0]

        def inner(j, carry2):
            slot = w * SCAP + j
            s = sid_ref[slot]
            g = gid_ref[slot]
            pltpu.make_async_copy(
                z_any.at[pl.ds(s, 1)], ztab.at[pl.ds(slot, 1)], sem_z).start()
            pltpu.make_async_copy(
                msg_any.at[pl.ds(g, 1)], mtab.at[pl.ds(slot, 1)], sem_m).start()
            return carry2

        return lax.fori_loop(0, c, inner, carry)

    lax.fori_loop(0, NW, outer_start, 0)

    c0.wait()
    c1.wait()

    def outer_wait(w, carry):
        c = cnt_ref[w, 0]

        def inner(j, carry2):
            slot = w * SCAP + j
            s = sid_ref[slot]
            g = gid_ref[slot]
            pltpu.make_async_copy(
                z_any.at[pl.ds(s, 1)], ztab.at[pl.ds(slot, 1)], sem_z).wait()
            pltpu.make_async_copy(
                msg_any.at[pl.ds(g, 1)], mtab.at[pl.ds(slot, 1)], sem_m).wait()
            return carry2

        return lax.fori_loop(0, c, inner, carry)

    lax.fori_loop(0, NW, outer_wait, 0)

    sum_ms = sums_ref[0, 0]
    ent = sums_ref[0, 1]

    te = jnp.cos(rel_ref[...] * wtime_ref[...])
    xcat = jnp.concatenate([ztab[...], te, mtab[...]], axis=1)
    m = jnp.dot(xcat, wmsg_ref[...], preferred_element_type=jnp.float32)
    m = jnp.maximum(m + bmsg_ref[...], 0.0)

    agg_s = jnp.dot(wts_ref[...], m, preferred_element_type=jnp.float32)
    agg_d = jnp.dot(wtd_ref[...], m, preferred_element_type=jnp.float32)

    z_s_in = jnp.concatenate([ztgt[0:1, :], agg_s], axis=1)
    z_d_in = jnp.concatenate([ztgt[1:2, :], agg_d], axis=1)
    z_s = jnp.maximum(jnp.dot(z_s_in, wupd_ref[...],
                              preferred_element_type=jnp.float32) + bupd_ref[...], 0.0)
    z_d = jnp.maximum(jnp.dot(z_d_in, wupd_ref[...],
                              preferred_element_type=jnp.float32) + bupd_ref[...], 0.0)

    h = jnp.concatenate([z_s, z_d], axis=1)
    h = jnp.maximum(jnp.dot(h, w1_ref[...],
                            preferred_element_type=jnp.float32) + b1_ref[...], 0.0)
    logits = jnp.dot(h, w2_ref[...],
                     preferred_element_type=jnp.float32) + b2_ref[...]

    feat = msg0_ref[:, NF:NF + NC]
    iota2 = lax.broadcasted_iota(jnp.int32, (1, NC), 1)
    mx = jnp.max(feat)
    lbl = jnp.min(jnp.where(feat == mx, iota2, NC))
    logit_lbl = jnp.sum(jnp.where(iota2 == lbl, logits, 0.0))

    lm = jnp.max(logits)
    lse = lm + jnp.log(jnp.sum(jnp.exp(logits - lm)))
    loss_pred = lse - logit_lbl

    total = loss_pred + 0.005 * sum_ms - 0.01 * ent
    out_ref[0, 0] = total


def kernel(z_original, last_update, edge_index, subgraph_t, subgraph_msg,
           edge_mask, w_time, W_msg, b_msg, W_upd, b_upd, W1, b1, W2, b2,
           target_src_local, target_dst_local):
    ts = jnp.asarray(target_src_local, jnp.int32)
    td = jnp.asarray(target_dst_local, jnp.int32)
    ts16 = jnp.full((16,), ts, jnp.int32)
    td16 = jnp.full((16,), td, jnp.int32)
    tstd = jnp.stack([ts, td])

    src = edge_index[0]
    dst = edge_index[1]

    cnt, sid, gid, wts, wtd, rel = _sc_find(
        src, dst, subgraph_t, edge_mask, last_update, ts16, td16)

    sums = pl.pallas_call(
        _tc_mask_body,
        out_shape=jax.ShapeDtypeStruct((1, 2), jnp.float32),
        out_specs=pl.BlockSpec(memory_space=pltpu.SMEM),
    )(edge_mask)

    vmem = pl.BlockSpec(memory_space=pltpu.VMEM)
    smem = pl.BlockSpec(memory_space=pltpu.SMEM)
    anym = pl.BlockSpec(memory_space=pl.ANY)

    total = pl.pallas_call(
        _tc_body,
        out_shape=jax.ShapeDtypeStruct((1, 1), jnp.float32),
        in_specs=[smem, smem, smem, smem, smem,
                  vmem, vmem, vmem, anym, anym,
                  vmem, vmem, vmem, vmem, vmem,
                  vmem, vmem, vmem, vmem, vmem],
        out_specs=smem,
        scratch_shapes=[
            pltpu.VMEM((TOT, D), jnp.float32),
            pltpu.VMEM((TOT, DM), jnp.float32),
            pltpu.VMEM((2, D), jnp.float32),
            pltpu.SemaphoreType.DMA,
            pltpu.SemaphoreType.DMA,
        ],
    )(
        sums,
        cnt,
        sid,
        gid,
        tstd,
        wts.reshape(1, TOT),
        wtd.reshape(1, TOT),
        rel.reshape(TOT, 1),
        z_original,
        subgraph_msg,
        w_time.reshape(1, TD),
        W_msg,
        b_msg.reshape(1, D),
        W_upd,
        b_upd.reshape(1, D),
        W1,
        b1.reshape(1, D),
        W2,
        b2.reshape(1, NC),
        subgraph_msg[0:1, :],
    )
    return total[0, 0]


# speedup vs baseline: 1.6173x; 1.0062x over previous
import functools

import jax
import jax.numpy as jnp
from jax import lax
from jax.experimental import pallas as pl
from jax.experimental.pallas import tpu as pltpu
from jax.experimental.pallas import tpu_sc as plsc

N = 10000
E = 320000
D = 128
TD = 32
DM = 64
NC = 32
NF = 16

NUM_SC_CORES = 2
NUM_SUBCORES = 16
NW = NUM_SC_CORES * NUM_SUBCORES
CH = E // NW
SEG = CH // 16
LCAP = 8
SLOTS = 16 * LCAP
SCAP = 16
TOT = NW * SCAP


def _sc_body(src_hbm, dst_hbm, t_hbm, mask_hbm, lu_hbm, ts_hbm, td_hbm,
             cnt_out, sid_out, gid_out, wts_out, wtd_out, rel_out,
             dst_v, src_v, t_v, mask_v, lu_v, ts_v, td_v,
             ids_v, pfx_v, cnt16_v,
             sids_v, gids_v, wtss_v, wtds_v, rels_v):
    wid = lax.axis_index("s") * NUM_SC_CORES + lax.axis_index("c")
    base = wid * CH

    pltpu.sync_copy(dst_hbm.at[pl.ds(base, CH)], dst_v)
    pltpu.sync_copy(src_hbm.at[pl.ds(base, CH)], src_v)
    pltpu.sync_copy(t_hbm.at[pl.ds(base, CH)], t_v)
    pltpu.sync_copy(mask_hbm.at[pl.ds(base, CH)], mask_v)
    pltpu.sync_copy(lu_hbm, lu_v)
    pltpu.sync_copy(ts_hbm, ts_v)
    pltpu.sync_copy(td_hbm, td_v)

    iota16 = lax.iota(jnp.int32, 16)
    tsv = ts_v[...]
    tdv = td_v[...]
    seg_base = iota16 * SEG
    lane_base = iota16 * LCAP

    def body(i, cnt):
        eid = seg_base + i
        d = plsc.load_gather(dst_v, [eid])
        m = (d == tsv) | (d == tdv)
        pos = lane_base + jnp.minimum(cnt, LCAP - 1)
        plsc.store_scatter(ids_v, [pos], eid, mask=m)
        return cnt + jnp.where(m, 1, 0).astype(jnp.int32)

    cnt = lax.fori_loop(0, SEG, body, jnp.zeros((16,), jnp.int32))
    cl = jnp.minimum(cnt, LCAP)

    pfx = cl
    for k in (1, 2, 4, 8):
        pfx_v[...] = pfx
        sh = plsc.load_gather(pfx_v, [jnp.maximum(iota16 - k, 0)])
        pfx = pfx + jnp.where(iota16 >= k, sh, 0)
    excl = pfx - cl
    pfx_v[...] = pfx
    tot = plsc.load_gather(pfx_v, [jnp.full((16,), 15, jnp.int32)])
    cnt16_v[...] = jnp.minimum(tot, SCAP)
    pltpu.sync_copy(cnt16_v, cnt_out.at[wid])

    zero_i = jnp.zeros((SCAP,), jnp.int32)
    zero_f = jnp.zeros((SCAP,), jnp.float32)
    sids_v[...] = zero_i
    gids_v[...] = zero_i
    wtss_v[...] = zero_f
    wtds_v[...] = zero_f
    rels_v[...] = zero_f

    for b in range(LCAP):
        valid = cnt > b
        ids_raw = plsc.load_gather(ids_v, [lane_base + b])
        ids = jnp.maximum(jnp.minimum(ids_raw, CH - 1), 0)
        d = plsc.load_gather(dst_v, [ids])
        s = plsc.load_gather(src_v, [ids])
        tv = plsc.load_gather(t_v, [ids])
        mv = plsc.load_gather(mask_v, [ids])
        lu = plsc.load_gather(lu_v, [s])
        relv = tv - lu
        msv = 1.0 / (1.0 + jnp.exp(-mv))
        zero = jnp.zeros((16,), jnp.float32)
        wts = jnp.where(valid & (d == tsv), msv, zero)
        wtd = jnp.where(valid & (d == tdv), msv, zero)
        pos = jnp.minimum(excl + b, SCAP - 1)
        plsc.store_scatter(sids_v, [pos], s, mask=valid)
        plsc.store_scatter(gids_v, [pos], base + ids, mask=valid)
        plsc.store_scatter(wtss_v, [pos], wts, mask=valid)
        plsc.store_scatter(wtds_v, [pos], wtd, mask=valid)
        plsc.store_scatter(rels_v, [pos], relv, mask=valid)

    pltpu.sync_copy(sids_v, sid_out.at[pl.ds(wid * SCAP, SCAP)])
    pltpu.sync_copy(gids_v, gid_out.at[pl.ds(wid * SCAP, SCAP)])
    pltpu.sync_copy(wtss_v, wts_out.at[pl.ds(wid * SCAP, SCAP)])
    pltpu.sync_copy(wtds_v, wtd_out.at[pl.ds(wid * SCAP, SCAP)])
    pltpu.sync_copy(rels_v, rel_out.at[pl.ds(wid * SCAP, SCAP)])


_sc_find = functools.partial(
    pl.kernel,
    mesh=plsc.VectorSubcoreMesh(core_axis_name="c", subcore_axis_name="s"),
    compiler_params=pltpu.CompilerParams(
        needs_layout_passes=False, use_tc_tiling_on_sc=False),
    out_type=[
        jax.ShapeDtypeStruct((NW, 16), jnp.int32),
        jax.ShapeDtypeStruct((TOT,), jnp.int32),
        jax.ShapeDtypeStruct((TOT,), jnp.int32),
        jax.ShapeDtypeStruct((TOT,), jnp.float32),
        jax.ShapeDtypeStruct((TOT,), jnp.float32),
        jax.ShapeDtypeStruct((TOT,), jnp.float32),
    ],
    scratch_types=[
        pltpu.VMEM((CH,), jnp.int32),
        pltpu.VMEM((CH,), jnp.int32),
        pltpu.VMEM((CH,), jnp.float32),
        pltpu.VMEM((CH,), jnp.float32),
        pltpu.VMEM((N,), jnp.float32),
        pltpu.VMEM((16,), jnp.int32),
        pltpu.VMEM((16,), jnp.int32),
        pltpu.VMEM((SLOTS,), jnp.int32),
        pltpu.VMEM((16,), jnp.int32),
        pltpu.VMEM((16,), jnp.int32),
        pltpu.VMEM((SCAP,), jnp.int32),
        pltpu.VMEM((SCAP,), jnp.int32),
        pltpu.VMEM((SCAP,), jnp.float32),
        pltpu.VMEM((SCAP,), jnp.float32),
        pltpu.VMEM((SCAP,), jnp.float32),
    ],
)(_sc_body)


def _tc_mask_body(mask_ref, out_ref):
    x = mask_ref[...].reshape(E // 128, 128)
    ms = 1.0 / (1.0 + jnp.exp(-x))
    out_ref[0, 0] = jnp.sum(ms)
    out_ref[0, 1] = jnp.sum(
        ms * jnp.log(ms + 1e-8) + (1.0 - ms) * jnp.log(1.0 - ms + 1e-8))


def _tc_body(sums_ref, cnt_ref, sid_ref, gid_ref, tstd_ref,
             wts_ref, wtd_ref, rel_ref, z_any, msg_any,
             wtime_ref, wmsg_ref, bmsg_ref, wupd_ref, bupd_ref,
             w1_ref, b1_ref, w2_ref, b2_ref, out_ref,
             ztab, mtab, ztgt, msg0, sem_z, sem_m):
    ztab[...] = jnp.zeros((TOT, D), jnp.float32)
    mtab[...] = jnp.zeros((TOT, DM), jnp.float32)

    c0 = pltpu.make_async_copy(
        z_any.at[pl.ds(tstd_ref[0], 1)], ztgt.at[pl.ds(0, 1)], sem_z)
    c1 = pltpu.make_async_copy(
        z_any.at[pl.ds(tstd_ref[1], 1)], ztgt.at[pl.ds(1, 1)], sem_z)
    cm0 = pltpu.make_async_copy(msg_any.at[pl.ds(0, 1)], msg0, sem_m)
    c0.start()
    c1.start()
    cm0.start()

    def outer_start(w, carry):
        c = cnt_ref[w, 0]

        def inner(j, carry2):
            slot = w * SCAP + j
            s = sid_ref[slot]
            g = gid_ref[slot]
            pltpu.make_async_copy(
                z_any.at[pl.ds(s, 1)], ztab.at[pl.ds(slot, 1)], sem_z).start()
            pltpu.make_async_copy(
                msg_any.at[pl.ds(g, 1)], mtab.at[pl.ds(slot, 1)], sem_m).start()
            return carry2

        return lax.fori_loop(0, c, inner, carry)

    lax.fori_loop(0, NW, outer_start, 0)

    c0.wait()
    c1.wait()
    cm0.wait()

    def outer_wait(w, carry):
        c = cnt_ref[w, 0]

        def inner(j, carry2):
            slot = w * SCAP + j
            s = sid_ref[slot]
            g = gid_ref[slot]
            pltpu.make_async_copy(
                z_any.at[pl.ds(s, 1)], ztab.at[pl.ds(slot, 1)], sem_z).wait()
            pltpu.make_async_copy(
                msg_any.at[pl.ds(g, 1)], mtab.at[pl.ds(slot, 1)], sem_m).wait()
            return carry2

        return lax.fori_loop(0, c, inner, carry)

    lax.fori_loop(0, NW, outer_wait, 0)

    sum_ms = sums_ref[0, 0]
    ent = sums_ref[0, 1]

    te = jnp.cos(rel_ref[...] * wtime_ref[...])
    xcat = jnp.concatenate([ztab[...], te, mtab[...]], axis=1)
    m = jnp.dot(xcat, wmsg_ref[...], preferred_element_type=jnp.float32)
    m = jnp.maximum(m + bmsg_ref[...], 0.0)

    agg_s = jnp.dot(wts_ref[...], m, preferred_element_type=jnp.float32)
    agg_d = jnp.dot(wtd_ref[...], m, preferred_element_type=jnp.float32)

    z_s_in = jnp.concatenate([ztgt[0:1, :], agg_s], axis=1)
    z_d_in = jnp.concatenate([ztgt[1:2, :], agg_d], axis=1)
    z_s = jnp.maximum(jnp.dot(z_s_in, wupd_ref[...],
                              preferred_element_type=jnp.float32) + bupd_ref[...], 0.0)
    z_d = jnp.maximum(jnp.dot(z_d_in, wupd_ref[...],
                              preferred_element_type=jnp.float32) + bupd_ref[...], 0.0)

    h = jnp.concatenate([z_s, z_d], axis=1)
    h = jnp.maximum(jnp.dot(h, w1_ref[...],
                            preferred_element_type=jnp.float32) + b1_ref[...], 0.0)
    logits = jnp.dot(h, w2_ref[...],
                     preferred_element_type=jnp.float32) + b2_ref[...]

    feat = msg0[:, NF:NF + NC]
    iota2 = lax.broadcasted_iota(jnp.int32, (1, NC), 1)
    mx = jnp.max(feat)
    lbl = jnp.min(jnp.where(feat == mx, iota2, NC))
    logit_lbl = jnp.sum(jnp.where(iota2 == lbl, logits, 0.0))

    lm = jnp.max(logits)
    lse = lm + jnp.log(jnp.sum(jnp.exp(logits - lm)))
    loss_pred = lse - logit_lbl

    total = loss_pred + 0.005 * sum_ms - 0.01 * ent
    out_ref[0, 0] = total


def kernel(z_original, last_update, edge_index, subgraph_t, subgraph_msg,
           edge_mask, w_time, W_msg, b_msg, W_upd, b_upd, W1, b1, W2, b2,
           target_src_local, target_dst_local):
    ts = jnp.asarray(target_src_local, jnp.int32)
    td = jnp.asarray(target_dst_local, jnp.int32)
    ts16 = jnp.full((16,), ts, jnp.int32)
    td16 = jnp.full((16,), td, jnp.int32)
    tstd = jnp.stack([ts, td])

    src = edge_index[0]
    dst = edge_index[1]

    cnt, sid, gid, wts, wtd, rel = _sc_find(
        src, dst, subgraph_t, edge_mask, last_update, ts16, td16)

    sums = pl.pallas_call(
        _tc_mask_body,
        out_shape=jax.ShapeDtypeStruct((1, 2), jnp.float32),
        out_specs=pl.BlockSpec(memory_space=pltpu.SMEM),
    )(edge_mask)

    vmem = pl.BlockSpec(memory_space=pltpu.VMEM)
    smem = pl.BlockSpec(memory_space=pltpu.SMEM)
    anym = pl.BlockSpec(memory_space=pl.ANY)

    total = pl.pallas_call(
        _tc_body,
        out_shape=jax.ShapeDtypeStruct((1, 1), jnp.float32),
        in_specs=[smem, smem, smem, smem, smem,
                  vmem, vmem, vmem, anym, anym,
                  vmem, vmem, vmem, vmem, vmem,
                  vmem, vmem, vmem, vmem],
        out_specs=smem,
        scratch_shapes=[
            pltpu.VMEM((TOT, D), jnp.float32),
            pltpu.VMEM((TOT, DM), jnp.float32),
            pltpu.VMEM((2, D), jnp.float32),
            pltpu.VMEM((1, DM), jnp.float32),
            pltpu.SemaphoreType.DMA,
            pltpu.SemaphoreType.DMA,
        ],
    )(
        sums,
        cnt,
        sid,
        gid,
        tstd,
        wts.reshape(1, TOT),
        wtd.reshape(1, TOT),
        rel.reshape(TOT, 1),
        z_original,
        subgraph_msg,
        w_time.reshape(1, TD),
        W_msg,
        b_msg.reshape(1, D),
        W_upd,
        b_upd.reshape(1, D),
        W1,
        b1.reshape(1, D),
        W2,
        b2.reshape(1, NC),
    )
    return total[0, 0]
